# baseline (speedup 1.0000x reference)
import jax
import jax.numpy as jnp
from jax.experimental import pallas as pl

_N = 500000
_DIM = 32
_ROWS = 4000


def _gru_body(mail_ref, mts_ref, memts_ref, mem_ref, rh_ref,
              wih_ref, whh_ref, bih_ref, bhh_ref, tw_ref, tb_ref,
              out_ref):
    dt = mts_ref[...] - memts_ref[...]
    tf = jnp.cos(dt * tw_ref[...] + tb_ref[...])
    t_in = jnp.concatenate([mail_ref[...], tf], axis=1)
    gx = jnp.dot(t_in, wih_ref[...],
                 preferred_element_type=jnp.float32) + bih_ref[...]
    mem = mem_ref[...]
    gh = jnp.dot(mem, whh_ref[...],
                 preferred_element_type=jnp.float32) + bhh_ref[...]
    d = _DIM
    r = jax.nn.sigmoid(gx[:, 0:d] + gh[:, 0:d])
    z = jax.nn.sigmoid(gx[:, d:2 * d] + gh[:, d:2 * d])
    n = jnp.tanh(gx[:, 2 * d:3 * d] + r * gh[:, 2 * d:3 * d])
    out_ref[...] = (1.0 - z) * n + z * mem + rh_ref[...]


def kernel(mail, mail_ts, mem_ts, mem, rh, W_ih, W_hh, b_ih, b_hh, time_w, time_b):
    n = mail.shape[0]
    rows = _ROWS
    grid = (n // rows,)
    d = _DIM

    mts = mail_ts.reshape(n, 1)
    memts = mem_ts.reshape(n, 1)
    wih_t = W_ih.T
    whh_t = W_hh.T
    bih = b_ih.reshape(1, 3 * d)
    bhh = b_hh.reshape(1, 3 * d)
    tw = time_w.reshape(1, d)
    tb = time_b.reshape(1, d)

    row_spec = lambda w: pl.BlockSpec((rows, w), lambda i: (i, 0))
    full_spec = lambda a: pl.BlockSpec(a.shape, lambda i: (0, 0))

    return pl.pallas_call(
        _gru_body,
        grid=grid,
        in_specs=[
            row_spec(d),
            row_spec(1),
            row_spec(1),
            row_spec(d),
            row_spec(d),
            full_spec(wih_t),
            full_spec(whh_t),
            full_spec(bih),
            full_spec(bhh),
            full_spec(tw),
            full_spec(tb),
        ],
        out_specs=row_spec(d),
        out_shape=jax.ShapeDtypeStruct((n, d), jnp.float32),
    )(mail, mts, memts, mem, rh, wih_t, whh_t, bih, bhh, tw, tb)


# speedup vs baseline: 1.7129x; 1.7129x over previous
import jax
import jax.numpy as jnp
from jax.experimental import pallas as pl

_DIM = 32
_PACK = 4
_LANES = _PACK * _DIM
_BLK = 1000

_INV_2PI = 0.15915494309189535
_C0 = 0.99999944368
_C1 = -19.739034373
_C2 = 64.93061337
_C3 = -85.295970962
_C4 = 58.912555324
_C5 = -21.283021593


def _cos2pi(t):
    u = t * t
    return _C0 + u * (_C1 + u * (_C2 + u * (_C3 + u * (_C4 + u * _C5))))


def _gru_body(mts_ref, memts_ref, mail_ref, mem_ref, rh_ref,
              bw_ref, wih_ref, whh_ref, bih_ref, bhh_ref, tb_ref,
              out_ref):
    d = _LANES
    dt = mts_ref[...] - memts_ref[...]
    x = jnp.dot(dt, bw_ref[...], precision=jax.lax.Precision.HIGHEST,
                preferred_element_type=jnp.float32) + tb_ref[...]
    t = x * _INV_2PI
    t = t - jnp.round(t)
    tf = _cos2pi(t)
    t_in = jnp.concatenate(
        [mail_ref[...].astype(jnp.bfloat16), tf.astype(jnp.bfloat16)],
        axis=1)
    gx = jnp.dot(t_in, wih_ref[...],
                 preferred_element_type=jnp.float32) + bih_ref[...]
    mem = mem_ref[...]
    gh = jnp.dot(mem.astype(jnp.bfloat16), whh_ref[...],
                 preferred_element_type=jnp.float32) + bhh_ref[...]
    r = jax.nn.sigmoid(gx[:, 0:d] + gh[:, 0:d])
    z = jax.nn.sigmoid(gx[:, d:2 * d] + gh[:, d:2 * d])
    n = jnp.tanh(gx[:, 2 * d:3 * d] + r * gh[:, 2 * d:3 * d])
    out_ref[...] = (1.0 - z) * n + z * mem + rh_ref[...]


def kernel(mail, mail_ts, mem_ts, mem, rh, W_ih, W_hh, b_ih, b_hh, time_w, time_b):
    n = mail.shape[0]
    d = _DIM
    p = _PACK
    lanes = _LANES
    np_ = n // p
    grid = (np_ // _BLK,)

    mail4 = mail.reshape(np_, lanes)
    mem4 = mem.reshape(np_, lanes)
    rh4 = rh.reshape(np_, lanes)
    mts4 = mail_ts.reshape(np_, p)
    memts4 = mem_ts.reshape(np_, p)

    eye = jnp.eye(p, dtype=jnp.float32)
    bw = jnp.kron(eye, time_w.reshape(1, d))
    tb4 = jnp.tile(time_b, p).reshape(1, lanes)

    wih_t = W_ih.T.reshape(2, d, 3, d)
    wih4 = jnp.einsum('pitj,gh->pgithj', wih_t, eye)
    wih4 = wih4.reshape(2 * lanes, 3 * lanes).astype(jnp.bfloat16)
    whh_t = W_hh.T.reshape(d, 3, d)
    whh4 = jnp.einsum('itj,gh->githj', whh_t, eye)
    whh4 = whh4.reshape(lanes, 3 * lanes).astype(jnp.bfloat16)
    bih4 = jnp.broadcast_to(b_ih.reshape(3, 1, d), (3, p, d)).reshape(1, 3 * lanes)
    bhh4 = jnp.broadcast_to(b_hh.reshape(3, 1, d), (3, p, d)).reshape(1, 3 * lanes)

    row_spec = lambda w: pl.BlockSpec((_BLK, w), lambda i: (i, 0))
    full_spec = lambda a: pl.BlockSpec(a.shape, lambda i: (0, 0))

    out = pl.pallas_call(
        _gru_body,
        grid=grid,
        in_specs=[
            row_spec(p),
            row_spec(p),
            row_spec(lanes),
            row_spec(lanes),
            row_spec(lanes),
            full_spec(bw),
            full_spec(wih4),
            full_spec(whh4),
            full_spec(bih4),
            full_spec(bhh4),
            full_spec(tb4),
        ],
        out_specs=row_spec(lanes),
        out_shape=jax.ShapeDtypeStruct((np_, lanes), jnp.float32),
    )(mts4, memts4, mail4, mem4, rh4, bw, wih4, whh4, bih4, bhh4, tb4)
    return out.reshape(n, d)
